# baseline (speedup 1.0000x reference)
import functools

import jax
import jax.numpy as jnp
from jax import lax
from jax.experimental import pallas as pl
from jax.experimental.pallas import tpu as pltpu
from jax.experimental.pallas import tpu_sc as plsc

_B, _S, _C = 4096, 50, 1000
_R = _B * _S
_NC = 2
_NS = 16
_NW = _NC * _NS
_RW = _R // _NW
_CR = 80
_NCHUNK = _RW // _CR
_NG = _CR // 16

_mesh = plsc.VectorSubcoreMesh(core_axis_name="c", subcore_axis_name="s")


@functools.partial(
    pl.kernel,
    mesh=_mesh,
    out_type=jax.ShapeDtypeStruct((_R, _C), jnp.int32),
    scratch_types=[
        pltpu.VMEM((_RW,), jnp.int32),
        pltpu.VMEM((_CR, _C), jnp.int32),
    ],
    compiler_params=pltpu.CompilerParams(needs_layout_passes=False),
)
def _sc_onehot(x_hbm, z_hbm, out_hbm, idx_v, buf_v):
    wid = lax.axis_index("s") * _NC + lax.axis_index("c")
    rbase = wid * _RW
    pltpu.sync_copy(x_hbm.at[pl.ds(rbase, _RW)], idx_v)
    pltpu.sync_copy(z_hbm, buf_v)

    lanes = lax.iota(jnp.int32, 16)
    ones = jnp.ones((16,), jnp.int32)
    zeros = jnp.zeros((16,), jnp.int32)

    def scat(c, val):
        def g_body(g, carry):
            lr = g * 16 + lanes
            xv = idx_v[pl.ds(c * _CR + g * 16, 16)]
            plsc.store_scatter(buf_v, [lr, xv], val)
            return carry

        lax.fori_loop(0, _NG, g_body, 0)

    def chunk(c, carry):
        scat(c, ones)
        pltpu.sync_copy(buf_v, out_hbm.at[pl.ds(rbase + c * _CR, _CR)])
        scat(c, zeros)
        return carry

    lax.fori_loop(0, _NCHUNK, chunk, 0)


def kernel(x):
    out = _sc_onehot(x.reshape(_R), jnp.zeros((_CR, _C), jnp.int32))
    return out.reshape(_B, _S, _C)


# speedup vs baseline: 1.3577x; 1.3577x over previous
import functools

import jax
import jax.numpy as jnp
from jax import lax
from jax.experimental import pallas as pl
from jax.experimental.pallas import tpu as pltpu
from jax.experimental.pallas import tpu_sc as plsc

_B, _S, _C = 4096, 50, 1000
_NC = 2
_NS = 16
_NW = _NC * _NS
_BPW = _B // _NW
_CB = 2
_RPC = _CB * _S
_NCHUNK = _BPW // _CB
_RW = _BPW * _S
_NG = -(-_RPC // 16)

_mesh = plsc.VectorSubcoreMesh(core_axis_name="c", subcore_axis_name="s")


@functools.partial(
    pl.kernel,
    mesh=_mesh,
    out_type=jax.ShapeDtypeStruct((_B, _S, _C), jnp.int32),
    scratch_types=[
        pltpu.VMEM((_RW + 16,), jnp.int32),
        pltpu.VMEM((_CB, _S, _C), jnp.int32),
    ],
    compiler_params=pltpu.CompilerParams(needs_layout_passes=False),
)
def _sc_onehot(x_hbm, z_hbm, out_hbm, idx_v, buf_v):
    wid = lax.axis_index("s") * _NC + lax.axis_index("c")
    bbase = wid * _BPW
    pltpu.sync_copy(x_hbm.at[pl.ds(bbase * _S, _RW)], idx_v.at[pl.ds(0, _RW)])
    pltpu.sync_copy(z_hbm, buf_v)

    lanes = lax.iota(jnp.int32, 16)
    ones = jnp.ones((16,), jnp.int32)
    zeros = jnp.zeros((16,), jnp.int32)

    def scat(c, val):
        def g_body(g, carry):
            j = g * 16 + lanes
            m = j < _RPC
            bi = jnp.where(j >= _S, 1, 0) + jnp.where(j >= 2 * _S, 1, 0)
            bi = jnp.minimum(bi, _CB - 1)
            si = j - bi * _S
            xv = idx_v[pl.ds(c * _RPC + g * 16, 16)]
            xv = jnp.minimum(jnp.maximum(xv, 0), _C - 1)
            plsc.store_scatter(buf_v, [bi, si, xv], val, mask=m)
            return carry

        lax.fori_loop(0, _NG, g_body, 0)

    def chunk(c, carry):
        scat(c, ones)
        pltpu.sync_copy(buf_v, out_hbm.at[pl.ds(bbase + c * _CB, _CB)])
        scat(c, zeros)
        return carry

    lax.fori_loop(0, _NCHUNK, chunk, 0)


def kernel(x):
    return _sc_onehot(x.reshape(_B * _S), jnp.zeros((_CB, _S, _C), jnp.int32))


# speedup vs baseline: 4.7967x; 3.5330x over previous
import functools

import jax
import jax.numpy as jnp
from jax import lax
from jax.experimental import pallas as pl
from jax.experimental.pallas import tpu as pltpu
from jax.experimental.pallas import tpu_sc as plsc

_B, _S, _C = 4096, 50, 1000
_NC = 2
_NS = 16
_NW = _NC * _NS
_BW = _B // _NW
_NG = _BW // 16

_mesh = plsc.VectorSubcoreMesh(core_axis_name="c", subcore_axis_name="s")


@functools.partial(
    pl.kernel,
    mesh=_mesh,
    out_type=jax.ShapeDtypeStruct((_S, _C, _B), jnp.int32),
    scratch_types=[
        pltpu.VMEM((_BW,), jnp.int32),
        pltpu.VMEM((1, _C, _BW), jnp.int32),
    ],
    compiler_params=pltpu.CompilerParams(needs_layout_passes=False),
)
def _sc_onehot(xt_hbm, z_hbm, out_hbm, idx_v, buf_v):
    wid = lax.axis_index("s") * _NC + lax.axis_index("c")
    b0 = wid * _BW
    pltpu.sync_copy(z_hbm, buf_v)

    lanes = lax.iota(jnp.int32, 16)
    ones = jnp.ones((16,), jnp.int32)
    zeros = jnp.zeros((16,), jnp.int32)
    zero16 = jnp.zeros((16,), jnp.int32)

    def scat(val):
        def g_body(g, carry):
            bi = g * 16 + lanes
            xv = idx_v[pl.ds(g * 16, 16)]
            xv = jnp.minimum(jnp.maximum(xv, 0), _C - 1)
            plsc.store_scatter(buf_v, [zero16, xv, bi], val)
            return carry

        lax.fori_loop(0, _NG, g_body, 0)

    def plane(s, carry):
        pltpu.sync_copy(xt_hbm.at[pl.ds(s * _B + b0, _BW)], idx_v)
        scat(ones)
        pltpu.sync_copy(
            buf_v, out_hbm.at[pl.ds(s, 1), pl.ds(0, _C), pl.ds(b0, _BW)]
        )
        scat(zeros)
        return carry

    lax.fori_loop(0, _S, plane, 0)


def kernel(x):
    xt = x.T.reshape(_S * _B)
    out_t = _sc_onehot(xt, jnp.zeros((1, _C, _BW), jnp.int32))
    return jnp.transpose(out_t, (2, 0, 1))


# speedup vs baseline: 5.2279x; 1.0899x over previous
import functools

import jax
import jax.numpy as jnp
from jax import lax
from jax.experimental import pallas as pl
from jax.experimental.pallas import tpu as pltpu
from jax.experimental.pallas import tpu_sc as plsc

_B, _S, _C = 4096, 50, 1000
_NC = 2
_NS = 16
_NW = _NC * _NS
_BW = _B // _NW
_NG = _BW // 16

_mesh = plsc.VectorSubcoreMesh(core_axis_name="c", subcore_axis_name="s")


@functools.partial(
    pl.kernel,
    mesh=_mesh,
    out_type=jax.ShapeDtypeStruct((_S, _C, _B), jnp.int32),
    scratch_types=[
        pltpu.VMEM((2, _BW), jnp.int32),
        pltpu.VMEM((1, _C, _BW), jnp.int32),
        pltpu.SemaphoreType.DMA,
    ],
    compiler_params=pltpu.CompilerParams(needs_layout_passes=False),
)
def _sc_onehot(xt_hbm, z_hbm, out_hbm, idx_v, buf_v, sem):
    wid = lax.axis_index("s") * _NC + lax.axis_index("c")
    b0 = wid * _BW
    pltpu.sync_copy(z_hbm, buf_v)
    pltpu.sync_copy(xt_hbm.at[pl.ds(b0, _BW)], idx_v.at[0])

    lanes = lax.iota(jnp.int32, 16)
    ones = jnp.ones((16,), jnp.int32)
    zeros = jnp.zeros((16,), jnp.int32)
    zero16 = jnp.zeros((16,), jnp.int32)

    def scat(slot, val):
        def g_body(g, carry):
            bi = g * 16 + lanes
            xv = idx_v[slot, pl.ds(g * 16, 16)]
            xv = jnp.minimum(jnp.maximum(xv, 0), _C - 1)
            plsc.store_scatter(buf_v, [zero16, xv, bi], val)
            return carry

        lax.fori_loop(0, _NG, g_body, 0)

    def plane(s, carry):
        slot = lax.rem(s, 2)
        nxt = lax.rem(s + 1, _S)
        nxt_dma = pltpu.make_async_copy(
            xt_hbm.at[pl.ds(nxt * _B + b0, _BW)],
            idx_v.at[lax.rem(s + 1, 2)],
            sem,
        )
        nxt_dma.start()
        scat(slot, ones)
        pltpu.sync_copy(
            buf_v, out_hbm.at[pl.ds(s, 1), pl.ds(0, _C), pl.ds(b0, _BW)]
        )
        scat(slot, zeros)
        nxt_dma.wait()
        return carry

    lax.fori_loop(0, _S, plane, 0)


def kernel(x):
    xt = x.T.reshape(_S * _B)
    out_t = _sc_onehot(xt, jnp.zeros((1, _C, _BW), jnp.int32))
    return jnp.transpose(out_t, (2, 0, 1))
